# baseline (speedup 1.0000x reference)
import functools

import jax
import jax.numpy as jnp
from jax import lax
from jax.experimental import pallas as pl
from jax.experimental.pallas import tpu as pltpu
from jax.experimental.pallas import tpu_sc as plsc

FEATURE_DIM = 128
MEMORY_SIZE = 100000
N_CLASSES = 1000
BATCH = 16384

BLK = 1024
C_PAD = 1024
TC_FEAT_BLOCKS = BATCH // BLK
TC_MEM_BLOCKS = 11
TC_MEM_ROWS = TC_MEM_BLOCKS * BLK

NC = 2
NS = 16
NW = NC * NS
CHUNK = 128
SUPER = 256

TAIL_START = BATCH + TC_MEM_ROWS
TAIL_ROWS = MEMORY_SIZE - TAIL_START
N_MEM_SUPERS = TAIL_ROWS // SUPER
MEM_EXTRA = N_MEM_SUPERS % NW
REM_BASE = TAIL_START + N_MEM_SUPERS * SUPER
REM = TAIL_ROWS - N_MEM_SUPERS * SUPER
assert REM == 160


def _sc_segsum(mem_features, mem_labels):
    mesh = plsc.VectorSubcoreMesh(core_axis_name="c", subcore_axis_name="s")

    @functools.partial(
        pl.kernel,
        mesh=mesh,
        out_type=jax.ShapeDtypeStruct((NC, N_CLASSES, FEATURE_DIM), jnp.float32),
        scratch_types=[
            pltpu.VMEM((2, SUPER, FEATURE_DIM), jnp.float32),
            pltpu.VMEM((2, 2, CHUNK), jnp.int32),
            pltpu.VMEM((32,), jnp.int32),
            pltpu.VMEM((64, FEATURE_DIM), jnp.float32),
            pltpu.VMEM_SHARED((N_CLASSES, FEATURE_DIM), jnp.float32),
            pltpu.SemaphoreType.DMA,
            pltpu.SemaphoreType.DMA,
            pltpu.SemaphoreType.DMA,
            pltpu.SemaphoreType.DMA,
        ],
    )
    def k(mem_hbm, mlab_hbm, out_hbm,
          rows_v, labs_v, idx_t, zbuf, shared, semf0, semf1, sems0, sems1):
        c = lax.axis_index("c")
        s = lax.axis_index("s")
        wid = s * NC + c
        sems = (semf0, semf1)
        sems_s = (sems0, sems1)

        n_t = N_MEM_SUPERS // NW + jnp.where(wid < MEM_EXTRA, 1, 0)

        def fetch(t, b):
            base = TAIL_START + (wid + t * NW) * SUPER
            pltpu.async_copy(mem_hbm.at[pl.ds(base, SUPER)],
                             rows_v.at[b], sems[b])
            pltpu.async_copy(mlab_hbm.at[pl.ds(base, CHUNK)],
                             labs_v.at[b, 0], sems[b])
            pltpu.async_copy(mlab_hbm.at[pl.ds(base + CHUNK, CHUNK)],
                             labs_v.at[b, 1], sems[b])

        def wait_fetch(b):
            pltpu.make_async_copy(mem_hbm.at[pl.ds(0, SUPER)],
                                  rows_v.at[b], sems[b]).wait()
            pltpu.make_async_copy(mlab_hbm.at[pl.ds(0, CHUNK)],
                                  labs_v.at[b, 0], sems[b]).wait()
            pltpu.make_async_copy(mlab_hbm.at[pl.ds(0, CHUNK)],
                                  labs_v.at[b, 1], sems[b]).wait()

        def scatter(b):
            for kk in range(SUPER // CHUNK):
                pltpu.async_copy(rows_v.at[b, pl.ds(kk * CHUNK, CHUNK)],
                                 shared.at[labs_v.at[b, kk]], sems_s[b],
                                 add=True)

        def wait_scatter(b):
            for kk in range(SUPER // CHUNK):
                pltpu.make_async_copy(rows_v.at[b, pl.ds(kk * CHUNK, CHUNK)],
                                      shared.at[labs_v.at[b, kk]],
                                      sems_s[b]).wait()

        fetch(jnp.int32(0), 0)

        zero16 = jnp.zeros((16,), jnp.float32)

        def zrow(r, carry):
            for cc in range(FEATURE_DIM // 16):
                zbuf[r, pl.ds(cc * 16, 16)] = zero16
            return carry

        lax.fori_loop(0, 64, zrow, 0)

        @pl.when(s < NS - 1)
        def _():
            pltpu.sync_copy(zbuf, shared.at[pl.ds(s * 64, 64)])

        @pl.when(s == NS - 1)
        def _():
            pltpu.sync_copy(zbuf.at[pl.ds(0, 40)], shared.at[pl.ds(960, 40)])

        plsc.subcore_barrier()

        def step(t, carry):
            def half(b):
                wait_fetch(b)

                @pl.when(t + 1 < n_t)
                def _():
                    @pl.when(t >= 1)
                    def _():
                        wait_scatter(1 - b)

                    fetch(t + 1, 1 - b)

                scatter(b)

            @pl.when(t % 2 == 0)
            def _():
                half(0)

            @pl.when(t % 2 == 1)
            def _():
                half(1)

            return carry

        lax.fori_loop(0, n_t, step, 0)
        wait_scatter(0)
        wait_scatter(1)

        @pl.when(wid == 6)
        def _():
            pltpu.sync_copy(mem_hbm.at[pl.ds(REM_BASE, CHUNK)],
                            rows_v.at[0, pl.ds(0, CHUNK)])
            pltpu.sync_copy(mlab_hbm.at[pl.ds(REM_BASE, CHUNK)],
                            labs_v.at[0, 0])
            pltpu.sync_copy(rows_v.at[0, pl.ds(0, CHUNK)],
                            shared.at[labs_v.at[0, 0]], add=True)

        @pl.when(wid == 7)
        def _():
            pltpu.sync_copy(mem_hbm.at[pl.ds(REM_BASE + CHUNK, 32)],
                            rows_v.at[0, pl.ds(0, 32)])
            pltpu.sync_copy(mlab_hbm.at[pl.ds(REM_BASE + CHUNK, 32)], idx_t)
            pltpu.sync_copy(rows_v.at[0, pl.ds(0, 32)],
                            shared.at[idx_t], add=True)

        plsc.subcore_barrier()

        @pl.when(s == 0)
        def _():
            pltpu.sync_copy(shared, out_hbm.at[c])

    return k(mem_features, mem_labels)


def _tc_body(f_ref, l_ref, m_ref, ml_ref, out_ref):
    i = pl.program_id(0)
    is_feat = i < TC_FEAT_BLOCKS
    rows = jnp.where(is_feat, f_ref[...], m_ref[...])
    lab = jnp.where(is_feat, l_ref[...], ml_ref[...])
    oht = (lax.broadcasted_iota(jnp.int16, (C_PAD, BLK), 0)
           == lab.astype(jnp.int16)[None, :]).astype(jnp.bfloat16)
    part = lax.dot_general(oht, rows.astype(jnp.bfloat16),
                           (((1,), (0,)), ((), ())),
                           preferred_element_type=jnp.float32)

    @pl.when(i == 0)
    def _():
        out_ref[...] = part

    @pl.when(i > 0)
    def _():
        out_ref[...] += part


def _tc_segsum(features, labels, mem_features, mem_labels):
    nf = TC_FEAT_BLOCKS
    return pl.pallas_call(
        _tc_body,
        grid=(TC_FEAT_BLOCKS + TC_MEM_BLOCKS,),
        in_specs=[
            pl.BlockSpec((BLK, FEATURE_DIM),
                         lambda i: (jnp.minimum(i, nf - 1), 0)),
            pl.BlockSpec((BLK,),
                         lambda i: (jnp.minimum(i, nf - 1),)),
            pl.BlockSpec((BLK, FEATURE_DIM),
                         lambda i: (jnp.maximum(i, nf), 0)),
            pl.BlockSpec((BLK,),
                         lambda i: (jnp.maximum(i, nf),)),
        ],
        out_specs=pl.BlockSpec((C_PAD, FEATURE_DIM), lambda i: (0, 0)),
        out_shape=jax.ShapeDtypeStruct((C_PAD, FEATURE_DIM), jnp.float32),
    )(features, labels, mem_features, mem_labels)


def _norm_body(p_ref, t_ref, o_ref):
    w = p_ref[0] + p_ref[1] + t_ref[pl.ds(0, N_CLASSES), :]
    nrm = jnp.sqrt(jnp.sum(w * w, axis=1, keepdims=True))
    o_ref[...] = w / jnp.maximum(nrm, 1e-12)


def _combine(sc_partials, tc_partial):
    return pl.pallas_call(
        _norm_body,
        out_shape=jax.ShapeDtypeStruct((N_CLASSES, FEATURE_DIM), jnp.float32),
    )(sc_partials, tc_partial)


def kernel(features, labels, mem_features, mem_labels):
    sc_partials = _sc_segsum(mem_features, mem_labels)
    tc_partial = _tc_segsum(features, labels, mem_features, mem_labels)
    return _combine(sc_partials, tc_partial)


# speedup vs baseline: 1.0188x; 1.0188x over previous
import functools

import jax
import jax.numpy as jnp
from jax import lax
from jax.experimental import pallas as pl
from jax.experimental.pallas import tpu as pltpu
from jax.experimental.pallas import tpu_sc as plsc

FEATURE_DIM = 128
MEMORY_SIZE = 100000
N_CLASSES = 1000
BATCH = 16384

BLK = 1024
C_PAD = 1024
TC_FEAT_BLOCKS = BATCH // BLK
TC_MEM_BLOCKS = 11
TC_MEM_ROWS = TC_MEM_BLOCKS * BLK

NC = 2
NS = 16
NW = NC * NS
CHUNK = 128
SUPER = 256

TAIL_START = BATCH + TC_MEM_ROWS
TAIL_ROWS = MEMORY_SIZE - TAIL_START
N_MEM_SUPERS = TAIL_ROWS // SUPER
MEM_EXTRA = N_MEM_SUPERS % NW
REM_BASE = TAIL_START + N_MEM_SUPERS * SUPER
REM = TAIL_ROWS - N_MEM_SUPERS * SUPER
assert REM == 160


def _sc_segsum(mem_features, mem_labels):
    mesh = plsc.VectorSubcoreMesh(core_axis_name="c", subcore_axis_name="s")

    @functools.partial(
        pl.kernel,
        mesh=mesh,
        out_type=jax.ShapeDtypeStruct((NC, N_CLASSES, FEATURE_DIM), jnp.float32),
        scratch_types=[
            pltpu.VMEM((2, SUPER, FEATURE_DIM), jnp.float32),
            pltpu.VMEM((2, 2, CHUNK), jnp.int32),
            pltpu.VMEM((32,), jnp.int32),
            pltpu.VMEM((64, FEATURE_DIM), jnp.float32),
            pltpu.VMEM_SHARED((N_CLASSES, FEATURE_DIM), jnp.float32),
            pltpu.SemaphoreType.DMA,
            pltpu.SemaphoreType.DMA,
            pltpu.SemaphoreType.DMA,
            pltpu.SemaphoreType.DMA,
        ],
    )
    def k(mem_hbm, mlab_hbm, out_hbm,
          rows_v, labs_v, idx_t, zbuf, shared, semf0, semf1, sems0, sems1):
        c = lax.axis_index("c")
        s = lax.axis_index("s")
        wid = s * NC + c
        sems = (semf0, semf1)
        sems_s = (sems0, sems1)

        n_t = N_MEM_SUPERS // NW + jnp.where(wid < MEM_EXTRA, 1, 0)

        def fetch(t, b):
            base = TAIL_START + (wid + t * NW) * SUPER
            pltpu.async_copy(mem_hbm.at[pl.ds(base, SUPER)],
                             rows_v.at[b], sems[b])
            pltpu.async_copy(mlab_hbm.at[pl.ds(base, CHUNK)],
                             labs_v.at[b, 0], sems[b])
            pltpu.async_copy(mlab_hbm.at[pl.ds(base + CHUNK, CHUNK)],
                             labs_v.at[b, 1], sems[b])

        def wait_fetch(b):
            pltpu.make_async_copy(mem_hbm.at[pl.ds(0, SUPER)],
                                  rows_v.at[b], sems[b]).wait()
            pltpu.make_async_copy(mlab_hbm.at[pl.ds(0, CHUNK)],
                                  labs_v.at[b, 0], sems[b]).wait()
            pltpu.make_async_copy(mlab_hbm.at[pl.ds(0, CHUNK)],
                                  labs_v.at[b, 1], sems[b]).wait()

        def scatter(b):
            for kk in range(SUPER // CHUNK):
                pltpu.async_copy(rows_v.at[b, pl.ds(kk * CHUNK, CHUNK)],
                                 shared.at[labs_v.at[b, kk]], sems_s[b],
                                 add=True)

        def wait_scatter(b):
            for kk in range(SUPER // CHUNK):
                pltpu.make_async_copy(rows_v.at[b, pl.ds(kk * CHUNK, CHUNK)],
                                      shared.at[labs_v.at[b, kk]],
                                      sems_s[b]).wait()

        fetch(jnp.int32(0), 0)

        zero16 = jnp.zeros((16,), jnp.float32)

        def zrow(r, carry):
            for cc in range(FEATURE_DIM // 16):
                zbuf[r, pl.ds(cc * 16, 16)] = zero16
            return carry

        lax.fori_loop(0, 64, zrow, 0)

        @pl.when(s < NS - 1)
        def _():
            pltpu.sync_copy(zbuf, shared.at[pl.ds(s * 64, 64)])

        @pl.when(s == NS - 1)
        def _():
            pltpu.sync_copy(zbuf.at[pl.ds(0, 40)], shared.at[pl.ds(960, 40)])

        plsc.subcore_barrier()

        def step(t, carry):
            def half(b):
                wait_fetch(b)

                @pl.when(t + 1 < n_t)
                def _():
                    @pl.when(t >= 1)
                    def _():
                        wait_scatter(1 - b)

                    fetch(t + 1, 1 - b)

                scatter(b)

            @pl.when(t % 2 == 0)
            def _():
                half(0)

            @pl.when(t % 2 == 1)
            def _():
                half(1)

            return carry

        lax.fori_loop(0, n_t, step, 0)
        wait_scatter(0)
        wait_scatter(1)

        @pl.when(wid == 6)
        def _():
            pltpu.sync_copy(mem_hbm.at[pl.ds(REM_BASE, CHUNK)],
                            rows_v.at[0, pl.ds(0, CHUNK)])
            pltpu.sync_copy(mlab_hbm.at[pl.ds(REM_BASE, CHUNK)],
                            labs_v.at[0, 0])
            pltpu.sync_copy(rows_v.at[0, pl.ds(0, CHUNK)],
                            shared.at[labs_v.at[0, 0]], add=True)

        @pl.when(wid == 7)
        def _():
            pltpu.sync_copy(mem_hbm.at[pl.ds(REM_BASE + CHUNK, 32)],
                            rows_v.at[0, pl.ds(0, 32)])
            pltpu.sync_copy(mlab_hbm.at[pl.ds(REM_BASE + CHUNK, 32)], idx_t)
            pltpu.sync_copy(rows_v.at[0, pl.ds(0, 32)],
                            shared.at[idx_t], add=True)

        plsc.subcore_barrier()

        @pl.when(s == 0)
        def _():
            pltpu.sync_copy(shared, out_hbm.at[c])

    return k(mem_features, mem_labels)


def _tc_body(f_ref, l_ref, m_ref, ml_ref, out_ref):
    i = pl.program_id(0)

    def accum(rows_ref, lab_ref):
        oht = (lax.broadcasted_iota(jnp.int32, (C_PAD, BLK), 0)
               == lab_ref[...][None, :]).astype(jnp.bfloat16)
        part = lax.dot_general(oht, rows_ref[...].astype(jnp.bfloat16),
                               (((1,), (0,)), ((), ())),
                               preferred_element_type=jnp.float32)

        @pl.when(i == 0)
        def _():
            out_ref[...] = part

        @pl.when(i > 0)
        def _():
            out_ref[...] += part

    @pl.when(i < TC_FEAT_BLOCKS)
    def _():
        accum(f_ref, l_ref)

    @pl.when(i >= TC_FEAT_BLOCKS)
    def _():
        accum(m_ref, ml_ref)


def _tc_segsum(features, labels, mem_features, mem_labels):
    nf = TC_FEAT_BLOCKS
    return pl.pallas_call(
        _tc_body,
        grid=(TC_FEAT_BLOCKS + TC_MEM_BLOCKS,),
        in_specs=[
            pl.BlockSpec((BLK, FEATURE_DIM),
                         lambda i: (jnp.minimum(i, nf - 1), 0)),
            pl.BlockSpec((BLK,),
                         lambda i: (jnp.minimum(i, nf - 1),)),
            pl.BlockSpec((BLK, FEATURE_DIM),
                         lambda i: (jnp.maximum(i, nf), 0)),
            pl.BlockSpec((BLK,),
                         lambda i: (jnp.maximum(i, nf),)),
        ],
        out_specs=pl.BlockSpec((C_PAD, FEATURE_DIM), lambda i: (0, 0)),
        out_shape=jax.ShapeDtypeStruct((C_PAD, FEATURE_DIM), jnp.float32),
    )(features, labels, mem_features, mem_labels)


def _norm_body(p_ref, t_ref, o_ref):
    w = p_ref[0] + p_ref[1] + t_ref[pl.ds(0, N_CLASSES), :]
    nrm = jnp.sqrt(jnp.sum(w * w, axis=1, keepdims=True))
    o_ref[...] = w / jnp.maximum(nrm, 1e-12)


def _combine(sc_partials, tc_partial):
    return pl.pallas_call(
        _norm_body,
        out_shape=jax.ShapeDtypeStruct((N_CLASSES, FEATURE_DIM), jnp.float32),
    )(sc_partials, tc_partial)


def kernel(features, labels, mem_features, mem_labels):
    sc_partials = _sc_segsum(mem_features, mem_labels)
    tc_partial = _tc_segsum(features, labels, mem_features, mem_labels)
    return _combine(sc_partials, tc_partial)
